# baseline (speedup 1.0000x reference)
import functools

import jax
import jax.numpy as jnp
from jax import lax
from jax.experimental import pallas as pl
from jax.experimental.pallas import tpu as pltpu
from jax.experimental.pallas import tpu_sc as plsc

N = 10000
E = 320000
D = 128
DH = D // 2

NC = 2
NT = 16
CH = 128
NCHUNK = 160
E_PAD = NT * NCHUNK * CH
N_PAD = 10240
RPT = N_PAD // NT
CW = 16
BR = 1024


ZB = 128


def _agg_body(hlo_hbm, hhi_hbm, srcg_hbm, dstg_hbm,
              oflo_hbm, ofhi_hbm, oblo_hbm, obhi_hbm,
              idx_g, idx_s, rows_a, rows_b, buf,
              acc, sem_a, sem_b):
    cid = lax.axis_index("c")
    sid = lax.axis_index("s")

    zero = jnp.zeros((16,), jnp.float32)

    def _zrow(i, _):
        for j in range(DH // 16):
            buf[i, pl.ds(j * 16, 16)] = zero
        return ()

    lax.fori_loop(0, ZB, _zrow, ())

    def _zero_acc():
        for j in range(RPT // ZB):
            pltpu.sync_copy(buf, acc.at[pl.ds(sid * RPT + j * ZB, ZB)])
        plsc.subcore_barrier()

    def _one_pass(h_hbm, out_hbm):
        _zero_acc()

        def _pair(k, _):
            ca = pltpu.async_copy(h_hbm.at[idx_g.at[2 * k]], rows_a, sem_a)
            cb = pltpu.async_copy(h_hbm.at[idx_g.at[2 * k + 1]], rows_b, sem_b)
            ca.wait()
            pltpu.sync_copy(rows_a, acc.at[idx_s.at[2 * k]], add=True)
            cb.wait()
            pltpu.sync_copy(rows_b, acc.at[idx_s.at[2 * k + 1]], add=True)
            return ()

        lax.fori_loop(0, NCHUNK // 2, _pair, ())
        plsc.subcore_barrier()

        for j in range(RPT // ZB):
            r0 = sid * RPT + j * ZB
            pltpu.sync_copy(acc.at[pl.ds(r0, ZB)], rows_a)
            pltpu.sync_copy(rows_a, out_hbm.at[pl.ds(r0, ZB)])

    def _direction(g_hbm, s_hbm, olo_hbm, ohi_hbm):
        pltpu.sync_copy(g_hbm.at[sid], idx_g)
        pltpu.sync_copy(s_hbm.at[sid], idx_s)
        _one_pass(hlo_hbm, olo_hbm)
        _one_pass(hhi_hbm, ohi_hbm)

    @pl.when(cid == 0)
    def _():
        _direction(srcg_hbm, dstg_hbm, oflo_hbm, ofhi_hbm)

    @pl.when(cid == 1)
    def _():
        _direction(dstg_hbm, srcg_hbm, oblo_hbm, obhi_hbm)


_HALF = jax.ShapeDtypeStruct((N_PAD, DH), jnp.float32)

_agg = pl.kernel(
    _agg_body,
    out_type=[_HALF, _HALF, _HALF, _HALF],
    mesh=plsc.VectorSubcoreMesh(core_axis_name="c", subcore_axis_name="s"),
    scratch_types=[
        pltpu.VMEM((NCHUNK, CH), jnp.int32),
        pltpu.VMEM((NCHUNK, CH), jnp.int32),
        pltpu.VMEM((CH, DH), jnp.float32),
        pltpu.VMEM((CH, DH), jnp.float32),
        pltpu.VMEM((ZB, DH), jnp.float32),
        pltpu.VMEM_SHARED((N_PAD, DH), jnp.float32),
    ] + [pltpu.SemaphoreType.DMA] * 2,
    compiler_params=pltpu.CompilerParams(use_tc_tiling_on_sc=False),
)


def _cnt_body(srcg_hbm, dstg_hbm, ones_hbm, outf_hbm, outb_hbm,
              idx_s, ones_v, cbuf, acc, sem):
    cid = lax.axis_index("c")
    sid = lax.axis_index("s")

    pltpu.sync_copy(ones_hbm, ones_v)
    zero = jnp.zeros((16,), jnp.float32)

    def _zrow(i, _):
        cbuf[i, :] = zero
        return ()

    lax.fori_loop(0, RPT, _zrow, ())
    pltpu.sync_copy(cbuf, acc.at[pl.ds(sid * RPT, RPT)])
    plsc.subcore_barrier()

    def _direction(s_hbm, out_hbm):
        pltpu.sync_copy(s_hbm.at[sid], idx_s)

        def _step(i, _):
            pltpu.sync_copy(ones_v, acc.at[idx_s.at[i]], add=True)
            return ()

        lax.fori_loop(0, NCHUNK, _step, ())
        plsc.subcore_barrier()
        pltpu.sync_copy(acc.at[pl.ds(sid * RPT, RPT)], cbuf)
        pltpu.sync_copy(cbuf, out_hbm.at[pl.ds(sid * RPT, RPT)])

    @pl.when(cid == 0)
    def _():
        _direction(dstg_hbm, outf_hbm)

    @pl.when(cid == 1)
    def _():
        _direction(srcg_hbm, outb_hbm)


_cnt = pl.kernel(
    _cnt_body,
    out_type=[jax.ShapeDtypeStruct((N_PAD, CW), jnp.float32),
              jax.ShapeDtypeStruct((N_PAD, CW), jnp.float32)],
    mesh=plsc.VectorSubcoreMesh(core_axis_name="c", subcore_axis_name="s"),
    scratch_types=[
        pltpu.VMEM((NCHUNK, CH), jnp.int32),
        pltpu.VMEM((CH, CW), jnp.float32),
        pltpu.VMEM((RPT, CW), jnp.float32),
        pltpu.VMEM_SHARED((N_PAD, CW), jnp.float32),
        pltpu.SemaphoreType.DMA,
    ],
    compiler_params=pltpu.CompilerParams(use_tc_tiling_on_sc=False),
)


def _mm_body(hlo_ref, hhi_ref, sflo_ref, sfhi_ref, sblo_ref, sbhi_ref,
             cf_ref, cb_ref,
             wslo_ref, wshi_ref, wflo_ref, wfhi_ref, wblo_ref, wbhi_ref,
             b_ref, o_ref, *, relu):
    invf = 1.0 / jnp.maximum(cf_ref[:, :1], 1.0)
    invb = 1.0 / jnp.maximum(cb_ref[:, :1], 1.0)

    def mm(a, w):
        return jnp.dot(a, w[...], preferred_element_type=jnp.float32)

    acc = mm(hlo_ref[...], wslo_ref) + mm(hhi_ref[...], wshi_ref)
    acc = acc + mm(sflo_ref[...] * invf, wflo_ref) + mm(sfhi_ref[...] * invf, wfhi_ref)
    acc = acc + mm(sblo_ref[...] * invb, wblo_ref) + mm(sbhi_ref[...] * invb, wbhi_ref)
    acc = acc + b_ref[...]
    o_ref[...] = jnp.maximum(acc, 0.0) if relu else acc


def _mm(hlo, hhi, sflo, sfhi, sblo, sbhi, cf, cb, w, b, relu):
    wt = w.T
    halves = [wt[i * DH:(i + 1) * DH, :] for i in range(6)]
    row_spec = pl.BlockSpec((BR, DH), lambda i: (i, 0))
    cnt_spec = pl.BlockSpec((BR, CW), lambda i: (i, 0))
    w_spec = pl.BlockSpec((DH, D), lambda i: (0, 0))
    return pl.pallas_call(
        functools.partial(_mm_body, relu=relu),
        grid=(N_PAD // BR,),
        in_specs=[row_spec] * 6 + [cnt_spec] * 2 + [w_spec] * 6
                 + [pl.BlockSpec((1, D), lambda i: (0, 0))],
        out_specs=pl.BlockSpec((BR, D), lambda i: (i, 0)),
        out_shape=jax.ShapeDtypeStruct((N_PAD, D), jnp.float32),
    )(hlo, hhi, sflo, sfhi, sblo, sbhi, cf, cb, *halves, b.reshape(1, D))


def kernel(x, edge_index, W0, b0, W1, b1):
    src = edge_index[0].astype(jnp.int32)
    dst = edge_index[1].astype(jnp.int32)
    padn = E_PAD - E
    srcg = jnp.concatenate(
        [src, jnp.full((padn,), N, jnp.int32)]).reshape(NT, NCHUNK, CH)
    dstg = jnp.concatenate(
        [dst, jnp.full((padn,), N + 1, jnp.int32)]).reshape(NT, NCHUNK, CH)
    xp = jnp.concatenate(
        [x.astype(jnp.float32), jnp.zeros((N_PAD - N, D), jnp.float32)], axis=0)
    ones_c = jnp.ones((CH, CW), jnp.float32)

    cntf, cntb = _cnt(srcg, dstg, ones_c)

    xlo, xhi = xp[:, :DH], xp[:, DH:]
    sf0lo, sf0hi, sb0lo, sb0hi = _agg(xlo, xhi, srcg, dstg)
    h1 = _mm(xlo, xhi, sf0lo, sf0hi, sb0lo, sb0hi, cntf, cntb, W0, b0,
             relu=True)

    h1lo, h1hi = h1[:, :DH], h1[:, DH:]
    sf1lo, sf1hi, sb1lo, sb1hi = _agg(h1lo, h1hi, srcg, dstg)
    out = _mm(h1lo, h1hi, sf1lo, sf1hi, sb1lo, sb1hi, cntf, cntb, W1, b1,
              relu=False)
    return out[:N]


# speedup vs baseline: 2.2757x; 2.2757x over previous
import functools

import jax
import jax.numpy as jnp
from jax import lax
from jax.experimental import pallas as pl
from jax.experimental.pallas import tpu as pltpu
from jax.experimental.pallas import tpu_sc as plsc

N = 10000
E = 320000
D = 128
DH = D // 2

NC = 2
NT = 16
CH = 128
NCHUNK = 160
E_PAD = NT * NCHUNK * CH
N_PAD = 10240
RPT = N_PAD // NT
CW = 16
BR = 1000


ZB = 128


def _agg_body(hlo_hbm, hhi_hbm, srcgg_hbm, dstgg_hbm, srcgs_hbm, dstgs_hbm,
              oflo_hbm, ofhi_hbm, oblo_hbm, obhi_hbm,
              idx_g, idx_s, rows_a, rows_b, buf,
              acc, sem_a, sem_b):
    cid = lax.axis_index("c")
    sid = lax.axis_index("s")

    zero = jnp.zeros((16,), jnp.float32)

    def _zrow(i, _):
        for j in range(DH // 16):
            buf[i, pl.ds(j * 16, 16)] = zero
        return ()

    lax.fori_loop(0, ZB, _zrow, ())

    def _zero_acc():
        for j in range(RPT // ZB):
            pltpu.sync_copy(buf, acc.at[pl.ds(sid * RPT + j * ZB, ZB)])
        plsc.subcore_barrier()

    def _one_pass(h_hbm, out_hbm):
        _zero_acc()

        def _pair(k, _):
            ca = pltpu.async_copy(h_hbm.at[idx_g.at[2 * k]], rows_a, sem_a)
            cb = pltpu.async_copy(h_hbm.at[idx_g.at[2 * k + 1]], rows_b, sem_b)
            ca.wait()
            pltpu.sync_copy(rows_a, acc.at[idx_s.at[2 * k]], add=True)
            cb.wait()
            pltpu.sync_copy(rows_b, acc.at[idx_s.at[2 * k + 1]], add=True)
            return ()

        lax.fori_loop(0, NCHUNK // 2, _pair, ())
        plsc.subcore_barrier()

        for j in range(RPT // ZB):
            r0 = sid * RPT + j * ZB
            pltpu.sync_copy(acc.at[pl.ds(r0, ZB)], rows_a)
            pltpu.sync_copy(rows_a, out_hbm.at[pl.ds(r0, ZB)])

    def _direction(g_hbm, s_hbm, olo_hbm, ohi_hbm):
        pltpu.sync_copy(g_hbm.at[sid], idx_g)
        pltpu.sync_copy(s_hbm.at[sid], idx_s)
        _one_pass(hlo_hbm, olo_hbm)
        _one_pass(hhi_hbm, ohi_hbm)

    @pl.when(cid == 0)
    def _():
        _direction(srcgg_hbm, dstgs_hbm, oflo_hbm, ofhi_hbm)

    @pl.when(cid == 1)
    def _():
        _direction(dstgg_hbm, srcgs_hbm, oblo_hbm, obhi_hbm)


_HALF = jax.ShapeDtypeStruct((N_PAD, DH), jnp.float32)

_agg = pl.kernel(
    _agg_body,
    out_type=[_HALF, _HALF, _HALF, _HALF],
    mesh=plsc.VectorSubcoreMesh(core_axis_name="c", subcore_axis_name="s"),
    scratch_types=[
        pltpu.VMEM((NCHUNK, CH), jnp.int32),
        pltpu.VMEM((NCHUNK, CH), jnp.int32),
        pltpu.VMEM((CH, DH), jnp.float32),
        pltpu.VMEM((CH, DH), jnp.float32),
        pltpu.VMEM((ZB, DH), jnp.float32),
        pltpu.VMEM_SHARED((N_PAD, DH), jnp.float32),
    ] + [pltpu.SemaphoreType.DMA] * 2,
    compiler_params=pltpu.CompilerParams(use_tc_tiling_on_sc=False),
)


def _cnt_body(srcg_hbm, dstg_hbm, ones_hbm, outf_hbm, outb_hbm,
              idx_s, ones_v, cbuf, acc, sem):
    cid = lax.axis_index("c")
    sid = lax.axis_index("s")

    pltpu.sync_copy(ones_hbm, ones_v)
    zero = jnp.zeros((16,), jnp.float32)

    def _zrow(i, _):
        cbuf[i, :] = zero
        return ()

    lax.fori_loop(0, RPT, _zrow, ())
    pltpu.sync_copy(cbuf, acc.at[pl.ds(sid * RPT, RPT)])
    plsc.subcore_barrier()

    def _direction(s_hbm, out_hbm):
        pltpu.sync_copy(s_hbm.at[sid], idx_s)

        def _step(i, _):
            pltpu.sync_copy(ones_v, acc.at[idx_s.at[i]], add=True)
            return ()

        lax.fori_loop(0, NCHUNK, _step, ())
        plsc.subcore_barrier()
        pltpu.sync_copy(acc.at[pl.ds(sid * RPT, RPT)], cbuf)
        pltpu.sync_copy(cbuf, out_hbm.at[pl.ds(sid * RPT, RPT)])

    @pl.when(cid == 0)
    def _():
        _direction(dstg_hbm, outf_hbm)

    @pl.when(cid == 1)
    def _():
        _direction(srcg_hbm, outb_hbm)


_cnt = pl.kernel(
    _cnt_body,
    out_type=[jax.ShapeDtypeStruct((N_PAD, CW), jnp.float32),
              jax.ShapeDtypeStruct((N_PAD, CW), jnp.float32)],
    mesh=plsc.VectorSubcoreMesh(core_axis_name="c", subcore_axis_name="s"),
    scratch_types=[
        pltpu.VMEM((NCHUNK, CH), jnp.int32),
        pltpu.VMEM((CH, CW), jnp.float32),
        pltpu.VMEM((RPT, CW), jnp.float32),
        pltpu.VMEM_SHARED((N_PAD, CW), jnp.float32),
        pltpu.SemaphoreType.DMA,
    ],
    compiler_params=pltpu.CompilerParams(use_tc_tiling_on_sc=False),
)


def _mm_body(hlo_ref, hhi_ref, sflo_ref, sfhi_ref, sblo_ref, sbhi_ref,
             cf_ref, cb_ref,
             wslo_ref, wshi_ref, wflo_ref, wfhi_ref, wblo_ref, wbhi_ref,
             b_ref, o_ref, *, relu):
    invf = 1.0 / jnp.maximum(cf_ref[:, :1], 1.0)
    invb = 1.0 / jnp.maximum(cb_ref[:, :1], 1.0)

    def mm(a, w):
        return jnp.dot(a, w[...], preferred_element_type=jnp.float32)

    acc = mm(hlo_ref[...], wslo_ref) + mm(hhi_ref[...], wshi_ref)
    acc = acc + mm(sflo_ref[...] * invf, wflo_ref) + mm(sfhi_ref[...] * invf, wfhi_ref)
    acc = acc + mm(sblo_ref[...] * invb, wblo_ref) + mm(sbhi_ref[...] * invb, wbhi_ref)
    acc = acc + b_ref[...]
    o_ref[...] = jnp.maximum(acc, 0.0) if relu else acc


def _mm(hlo, hhi, sflo, sfhi, sblo, sbhi, cf, cb, w, b, relu):
    wt = w.T
    halves = [wt[i * DH:(i + 1) * DH, :] for i in range(6)]
    row_spec = pl.BlockSpec((BR, DH), lambda i: (i, 0))
    cnt_spec = pl.BlockSpec((BR, CW), lambda i: (i, 0))
    w_spec = pl.BlockSpec((DH, D), lambda i: (0, 0))
    return pl.pallas_call(
        functools.partial(_mm_body, relu=relu),
        grid=(N // BR,),
        in_specs=[row_spec] * 6 + [cnt_spec] * 2 + [w_spec] * 6
                 + [pl.BlockSpec((1, D), lambda i: (0, 0))],
        out_specs=pl.BlockSpec((BR, D), lambda i: (i, 0)),
        out_shape=jax.ShapeDtypeStruct((N, D), jnp.float32),
    )(hlo, hhi, sflo, sfhi, sblo, sbhi, cf, cb, *halves, b.reshape(1, D))


def kernel(x, edge_index, W0, b0, W1, b1):
    src = edge_index[0].astype(jnp.int32)
    dst = edge_index[1].astype(jnp.int32)
    padn = E_PAD - E
    ar = jnp.arange(padn, dtype=jnp.int32)
    dummy_g = ar * 97 % N
    dummy_s = N + ar % (N_PAD - N)
    srcg_g = jnp.concatenate([src, dummy_g]).reshape(NT, NCHUNK, CH)
    dstg_g = jnp.concatenate([dst, dummy_g]).reshape(NT, NCHUNK, CH)
    srcg_s = jnp.concatenate([src, dummy_s]).reshape(NT, NCHUNK, CH)
    dstg_s = jnp.concatenate([dst, dummy_s]).reshape(NT, NCHUNK, CH)
    ones_c = jnp.ones((CH, CW), jnp.float32)

    cntf, cntb = _cnt(srcg_s, dstg_s, ones_c)

    xf = x.astype(jnp.float32)
    xlo, xhi = xf[:, :DH], xf[:, DH:]
    sf0lo, sf0hi, sb0lo, sb0hi = _agg(xlo, xhi, srcg_g, dstg_g, srcg_s, dstg_s)
    h1 = _mm(xlo, xhi, sf0lo, sf0hi, sb0lo, sb0hi, cntf, cntb, W0, b0,
             relu=True)

    h1lo, h1hi = h1[:, :DH], h1[:, DH:]
    sf1lo, sf1hi, sb1lo, sb1hi = _agg(h1lo, h1hi, srcg_g, dstg_g, srcg_s,
                                      dstg_s)
    out = _mm(h1lo, h1hi, sf1lo, sf1hi, sb1lo, sb1hi, cntf, cntb, W1, b1,
              relu=False)
    return out


# speedup vs baseline: 2.6030x; 1.1438x over previous
import functools

import jax
import jax.numpy as jnp
from jax import lax
from jax.experimental import pallas as pl
from jax.experimental.pallas import tpu as pltpu
from jax.experimental.pallas import tpu_sc as plsc

N = 10000
E = 320000
D = 128
DH = D // 2

NC = 2
NT = 16
CH = 128
NCHUNK = 160
G = 4
E_PAD = NT * NCHUNK * CH
N_PAD = 10240
RPT = N_PAD // NT
CW = 16
BR = 1000


ZB = 128


def _agg_body(hlo_hbm, hhi_hbm, srcgg_hbm, dstgg_hbm, srcgs_hbm, dstgs_hbm,
              oflo_hbm, ofhi_hbm, oblo_hbm, obhi_hbm,
              idx_g, idx_s, rows0, rows1, rows2, rows3, buf,
              acc, g0, g1, g2, g3, s0, s1, s2, s3):
    rows = (rows0, rows1, rows2, rows3)
    gsem = (g0, g1, g2, g3)
    ssem = (s0, s1, s2, s3)
    cid = lax.axis_index("c")
    sid = lax.axis_index("s")

    zero = jnp.zeros((16,), jnp.float32)

    def _zrow(i, _):
        for j in range(DH // 16):
            buf[i, pl.ds(j * 16, 16)] = zero
        return ()

    lax.fori_loop(0, ZB, _zrow, ())

    def _zero_acc():
        for j in range(RPT // ZB):
            pltpu.sync_copy(buf, acc.at[pl.ds(sid * RPT + j * ZB, ZB)])
        plsc.subcore_barrier()

    def _one_pass(h_hbm, out_hbm):
        _zero_acc()

        def _grp(k, _):
            base = k * G
            gd = [pltpu.async_copy(h_hbm.at[idx_g.at[base + b]], rows[b],
                                   gsem[b]) for b in range(G)]
            sd = []
            for b in range(G):
                gd[b].wait()
                sd.append(pltpu.async_copy(
                    rows[b], acc.at[idx_s.at[base + b]], ssem[b], add=True))
            for b in range(G):
                sd[b].wait()
            return ()

        lax.fori_loop(0, NCHUNK // G, _grp, ())
        plsc.subcore_barrier()

        for j in range(RPT // ZB):
            r0 = sid * RPT + j * ZB
            pltpu.sync_copy(acc.at[pl.ds(r0, ZB)], rows0.at[pl.ds(0, ZB)])
            pltpu.sync_copy(rows0.at[pl.ds(0, ZB)], out_hbm.at[pl.ds(r0, ZB)])

    def _direction(g_hbm, s_hbm, olo_hbm, ohi_hbm):
        pltpu.sync_copy(g_hbm.at[sid], idx_g)
        pltpu.sync_copy(s_hbm.at[sid], idx_s)
        _one_pass(hlo_hbm, olo_hbm)
        _one_pass(hhi_hbm, ohi_hbm)

    @pl.when(cid == 0)
    def _():
        _direction(srcgg_hbm, dstgs_hbm, oflo_hbm, ofhi_hbm)

    @pl.when(cid == 1)
    def _():
        _direction(dstgg_hbm, srcgs_hbm, oblo_hbm, obhi_hbm)


_HALF = jax.ShapeDtypeStruct((N_PAD, DH), jnp.float32)

_agg = pl.kernel(
    _agg_body,
    out_type=[_HALF, _HALF, _HALF, _HALF],
    mesh=plsc.VectorSubcoreMesh(core_axis_name="c", subcore_axis_name="s"),
    scratch_types=[
        pltpu.VMEM((NCHUNK, CH), jnp.int32),
        pltpu.VMEM((NCHUNK, CH), jnp.int32),
        pltpu.VMEM((CH, DH), jnp.float32),
        pltpu.VMEM((CH, DH), jnp.float32),
        pltpu.VMEM((CH, DH), jnp.float32),
        pltpu.VMEM((CH, DH), jnp.float32),
        pltpu.VMEM((ZB, DH), jnp.float32),
        pltpu.VMEM_SHARED((N_PAD, DH), jnp.float32),
    ] + [pltpu.SemaphoreType.DMA] * 8,
    compiler_params=pltpu.CompilerParams(use_tc_tiling_on_sc=False),
)


def _cnt_body(srcg_hbm, dstg_hbm, ones_hbm, outf_hbm, outb_hbm,
              idx_s, ones_v, cbuf, acc, sem):
    cid = lax.axis_index("c")
    sid = lax.axis_index("s")

    pltpu.sync_copy(ones_hbm, ones_v)
    zero = jnp.zeros((16,), jnp.float32)

    def _zrow(i, _):
        cbuf[i, :] = zero
        return ()

    lax.fori_loop(0, RPT, _zrow, ())
    pltpu.sync_copy(cbuf, acc.at[pl.ds(sid * RPT, RPT)])
    plsc.subcore_barrier()

    def _direction(s_hbm, out_hbm):
        pltpu.sync_copy(s_hbm.at[sid], idx_s)

        def _step(i, _):
            pltpu.sync_copy(ones_v, acc.at[idx_s.at[i]], add=True)
            return ()

        lax.fori_loop(0, NCHUNK, _step, ())
        plsc.subcore_barrier()
        pltpu.sync_copy(acc.at[pl.ds(sid * RPT, RPT)], cbuf)
        pltpu.sync_copy(cbuf, out_hbm.at[pl.ds(sid * RPT, RPT)])

    @pl.when(cid == 0)
    def _():
        _direction(dstg_hbm, outf_hbm)

    @pl.when(cid == 1)
    def _():
        _direction(srcg_hbm, outb_hbm)


_cnt = pl.kernel(
    _cnt_body,
    out_type=[jax.ShapeDtypeStruct((N_PAD, CW), jnp.float32),
              jax.ShapeDtypeStruct((N_PAD, CW), jnp.float32)],
    mesh=plsc.VectorSubcoreMesh(core_axis_name="c", subcore_axis_name="s"),
    scratch_types=[
        pltpu.VMEM((NCHUNK, CH), jnp.int32),
        pltpu.VMEM((CH, CW), jnp.float32),
        pltpu.VMEM((RPT, CW), jnp.float32),
        pltpu.VMEM_SHARED((N_PAD, CW), jnp.float32),
        pltpu.SemaphoreType.DMA,
    ],
    compiler_params=pltpu.CompilerParams(use_tc_tiling_on_sc=False),
)


def _mm_body(hlo_ref, hhi_ref, sflo_ref, sfhi_ref, sblo_ref, sbhi_ref,
             cf_ref, cb_ref,
             wslo_ref, wshi_ref, wflo_ref, wfhi_ref, wblo_ref, wbhi_ref,
             b_ref, o_ref, *, relu):
    invf = 1.0 / jnp.maximum(cf_ref[:, :1], 1.0)
    invb = 1.0 / jnp.maximum(cb_ref[:, :1], 1.0)

    def mm(a, w):
        return jnp.dot(a, w[...], preferred_element_type=jnp.float32)

    acc = mm(hlo_ref[...], wslo_ref) + mm(hhi_ref[...], wshi_ref)
    acc = acc + mm(sflo_ref[...] * invf, wflo_ref) + mm(sfhi_ref[...] * invf, wfhi_ref)
    acc = acc + mm(sblo_ref[...] * invb, wblo_ref) + mm(sbhi_ref[...] * invb, wbhi_ref)
    acc = acc + b_ref[...]
    o_ref[...] = jnp.maximum(acc, 0.0) if relu else acc


def _mm(hlo, hhi, sflo, sfhi, sblo, sbhi, cf, cb, w, b, relu):
    wt = w.T
    halves = [wt[i * DH:(i + 1) * DH, :] for i in range(6)]
    row_spec = pl.BlockSpec((BR, DH), lambda i: (i, 0))
    cnt_spec = pl.BlockSpec((BR, CW), lambda i: (i, 0))
    w_spec = pl.BlockSpec((DH, D), lambda i: (0, 0))
    return pl.pallas_call(
        functools.partial(_mm_body, relu=relu),
        grid=(N // BR,),
        in_specs=[row_spec] * 6 + [cnt_spec] * 2 + [w_spec] * 6
                 + [pl.BlockSpec((1, D), lambda i: (0, 0))],
        out_specs=pl.BlockSpec((BR, D), lambda i: (i, 0)),
        out_shape=jax.ShapeDtypeStruct((N, D), jnp.float32),
    )(hlo, hhi, sflo, sfhi, sblo, sbhi, cf, cb, *halves, b.reshape(1, D))


def kernel(x, edge_index, W0, b0, W1, b1):
    src = edge_index[0].astype(jnp.int32)
    dst = edge_index[1].astype(jnp.int32)
    padn = E_PAD - E
    ar = jnp.arange(padn, dtype=jnp.int32)
    dummy_g = ar * 97 % N
    dummy_s = N + ar % (N_PAD - N)
    srcg_g = jnp.concatenate([src, dummy_g]).reshape(NT, NCHUNK, CH)
    dstg_g = jnp.concatenate([dst, dummy_g]).reshape(NT, NCHUNK, CH)
    srcg_s = jnp.concatenate([src, dummy_s]).reshape(NT, NCHUNK, CH)
    dstg_s = jnp.concatenate([dst, dummy_s]).reshape(NT, NCHUNK, CH)
    ones_c = jnp.ones((CH, CW), jnp.float32)

    cntf, cntb = _cnt(srcg_s, dstg_s, ones_c)

    xf = x.astype(jnp.float32)
    xlo, xhi = xf[:, :DH], xf[:, DH:]
    sf0lo, sf0hi, sb0lo, sb0hi = _agg(xlo, xhi, srcg_g, dstg_g, srcg_s, dstg_s)
    h1 = _mm(xlo, xhi, sf0lo, sf0hi, sb0lo, sb0hi, cntf, cntb, W0, b0,
             relu=True)

    h1lo, h1hi = h1[:, :DH], h1[:, DH:]
    sf1lo, sf1hi, sb1lo, sb1hi = _agg(h1lo, h1hi, srcg_g, dstg_g, srcg_s,
                                      dstg_s)
    out = _mm(h1lo, h1hi, sf1lo, sf1hi, sb1lo, sb1hi, cntf, cntb, W1, b1,
              relu=False)
    return out
